# baseline (speedup 1.0000x reference)
import functools

import jax
import jax.numpy as jnp
from jax import lax
from jax.experimental import pallas as pl
from jax.experimental.pallas import tpu as pltpu
from jax.experimental.pallas import tpu_sc as plsc

H = 128
L = 16
NC = 2
NS = 16
NW = NC * NS
B, S = 4, 4096
TOKENS = B * S
TOK_PER_W = TOKENS // NW
CHUNK = 128
NCHUNK = TOK_PER_W // CHUNK
EPS = 1e-6

TC_ROWS = 1024


def _gather_sum_body(wid_hbm, pid_hbm, wtab_hbm, ptab_hbm, out_hbm,
                     idxw_v, idxp_v, rows_w, rows_p, semw, semp):
    w = lax.axis_index("s") * NC + lax.axis_index("c")
    base_w = w * TOK_PER_W
    b_idx = w // (S // TOK_PER_W)
    col0 = (w % (S // TOK_PER_W)) * TOK_PER_W

    pltpu.sync_copy(wid_hbm.at[b_idx, pl.ds(col0, TOK_PER_W)], idxw_v)
    pltpu.sync_copy(pid_hbm.at[b_idx, pl.ds(col0, TOK_PER_W)], idxp_v)

    def start(c):
        b = c % 2
        cw = pltpu.async_copy(
            wtab_hbm.at[idxw_v.at[pl.ds(c * CHUNK, CHUNK)]], rows_w.at[b],
            semw)
        cp = pltpu.async_copy(
            ptab_hbm.at[idxp_v.at[pl.ds(c * CHUNK, CHUNK)]], rows_p.at[b],
            semp)
        return cw, cp

    pend = start(0)
    for c in range(NCHUNK):
        b = c % 2
        cw, cp = pend
        cw.wait()
        cp.wait()
        if c + 1 < NCHUNK:
            pend = start(c + 1)

        def body(t, carry):
            for j in range(H // L):
                sl = pl.ds(j * L, L)
                rows_w[b, t, sl] = rows_w[b, t, sl] + rows_p[b, t, sl]
            return carry

        lax.fori_loop(0, CHUNK, body, 0, unroll=4)
        pltpu.sync_copy(rows_w.at[b],
                        out_hbm.at[pl.ds(base_w + c * CHUNK, CHUNK)])


def _ln_tc_kernel(sum_ref, tid_ref, ttab_ref, gamma_ref, beta_ref, out_ref):
    x = sum_ref[...]
    tidf = tid_ref[...].astype(jnp.float32)
    t0 = ttab_ref[0:1, :]
    t1 = ttab_ref[1:2, :]
    x = x + t0 + tidf * (t1 - t0)
    mean = jnp.mean(x, axis=-1, keepdims=True)
    var = jnp.mean(jnp.square(x - mean), axis=-1, keepdims=True)
    normed = (x - mean) * lax.rsqrt(var + EPS)
    out_ref[...] = normed * gamma_ref[0, :] + beta_ref[0, :]


@jax.jit
def _run(word_ids, pos_ids, type_ids, word_table, pos_table, type_table,
         ln_gamma, ln_beta):
    mesh = plsc.VectorSubcoreMesh(core_axis_name="c", subcore_axis_name="s")
    sc_k = pl.kernel(
        _gather_sum_body,
        mesh=mesh,
        compiler_params=pltpu.CompilerParams(needs_layout_passes=False),
        out_type=jax.ShapeDtypeStruct((TOKENS, H), jnp.float32),
        scratch_types=[
            pltpu.VMEM((TOK_PER_W,), jnp.int32),
            pltpu.VMEM((TOK_PER_W,), jnp.int32),
            pltpu.VMEM((2, CHUNK, H), jnp.float32),
            pltpu.VMEM((2, CHUNK, H), jnp.float32),
            pltpu.SemaphoreType.DMA,
            pltpu.SemaphoreType.DMA,
        ],
    )
    summed = sc_k(word_ids, pos_ids, word_table, pos_table)

    ngrid = TOKENS // TC_ROWS
    tids = type_ids.reshape(TOKENS, 1)
    out = pl.pallas_call(
        _ln_tc_kernel,
        grid=(ngrid,),
        in_specs=[
            pl.BlockSpec((TC_ROWS, H), lambda i: (i, 0)),
            pl.BlockSpec((TC_ROWS, 1), lambda i: (i, 0)),
            pl.BlockSpec((2, H), lambda i: (0, 0)),
            pl.BlockSpec((1, H), lambda i: (0, 0)),
            pl.BlockSpec((1, H), lambda i: (0, 0)),
        ],
        out_specs=pl.BlockSpec((TC_ROWS, H), lambda i: (i, 0)),
        out_shape=jax.ShapeDtypeStruct((TOKENS, H), jnp.float32),
    )(summed, tids, type_table, ln_gamma.reshape(1, H),
      ln_beta.reshape(1, H))
    return out.reshape(B, S, H)


def kernel(word_ids, pos_ids, type_ids, word_table, pos_table, type_table,
           ln_gamma, ln_beta):
    return _run(word_ids, pos_ids, type_ids, word_table, pos_table,
                type_table, ln_gamma, ln_beta)


# speedup vs baseline: 1.0757x; 1.0757x over previous
import functools

import jax
import jax.numpy as jnp
from jax import lax
from jax.experimental import pallas as pl
from jax.experimental.pallas import tpu as pltpu
from jax.experimental.pallas import tpu_sc as plsc

H = 128
L = 16
NC = 2
NS = 16
NW = NC * NS
B, S = 4, 4096
TOKENS = B * S
TOK_PER_W = TOKENS // NW
CHUNK = 128
NCHUNK = TOK_PER_W // CHUNK
EPS = 1e-6

TC_ROWS = 2048


def _gather_sum_body(wid_hbm, pid_hbm, wtab_hbm, ptab_hbm, out_hbm,
                     idxw_v, idxp_v, rows_w, rows_p, semw, semp):
    w = lax.axis_index("s") * NC + lax.axis_index("c")
    base_w = w * TOK_PER_W
    b_idx = w // (S // TOK_PER_W)
    col0 = (w % (S // TOK_PER_W)) * TOK_PER_W

    pltpu.sync_copy(wid_hbm.at[b_idx, pl.ds(col0, TOK_PER_W)], idxw_v)
    pltpu.sync_copy(pid_hbm.at[b_idx, pl.ds(col0, TOK_PER_W)], idxp_v)

    def start(c):
        b = c % 2
        cw = pltpu.async_copy(
            wtab_hbm.at[idxw_v.at[pl.ds(c * CHUNK, CHUNK)]], rows_w.at[b],
            semw)
        cp = pltpu.async_copy(
            ptab_hbm.at[idxp_v.at[pl.ds(c * CHUNK, CHUNK)]], rows_p.at[b],
            semp)
        return cw, cp

    pend = start(0)
    for c in range(NCHUNK):
        b = c % 2
        cw, cp = pend
        cw.wait()
        cp.wait()
        if c + 1 < NCHUNK:
            pend = start(c + 1)

        def body(t, carry):
            for j in range(H // L):
                sl = pl.ds(j * L, L)
                rows_w[b, t, sl] = rows_w[b, t, sl] + rows_p[b, t, sl]
            return carry

        lax.fori_loop(0, CHUNK, body, 0, unroll=4)
        pltpu.sync_copy(rows_w.at[b],
                        out_hbm.at[pl.ds(base_w + c * CHUNK, CHUNK)])


def _ln_tc_kernel(sum_ref, tid_ref, ttab_ref, gamma_ref, beta_ref, out_ref):
    x = sum_ref[...]
    tidf = tid_ref[...].astype(jnp.float32)
    t0 = ttab_ref[0:1, :]
    t1 = ttab_ref[1:2, :]
    x = x + t0 + tidf * (t1 - t0)
    mean = jnp.mean(x, axis=-1, keepdims=True)
    var = jnp.mean(jnp.square(x - mean), axis=-1, keepdims=True)
    normed = (x - mean) * lax.rsqrt(var + EPS)
    out_ref[...] = normed * gamma_ref[0, :] + beta_ref[0, :]


@jax.jit
def _run(word_ids, pos_ids, type_ids, word_table, pos_table, type_table,
         ln_gamma, ln_beta):
    mesh = plsc.VectorSubcoreMesh(core_axis_name="c", subcore_axis_name="s")
    sc_k = pl.kernel(
        _gather_sum_body,
        mesh=mesh,
        compiler_params=pltpu.CompilerParams(needs_layout_passes=False),
        out_type=jax.ShapeDtypeStruct((TOKENS, H), jnp.float32),
        scratch_types=[
            pltpu.VMEM((TOK_PER_W,), jnp.int32),
            pltpu.VMEM((TOK_PER_W,), jnp.int32),
            pltpu.VMEM((2, CHUNK, H), jnp.float32),
            pltpu.VMEM((2, CHUNK, H), jnp.float32),
            pltpu.SemaphoreType.DMA,
            pltpu.SemaphoreType.DMA,
        ],
    )
    summed = sc_k(word_ids, pos_ids, word_table, pos_table)

    ngrid = TOKENS // TC_ROWS
    tids = type_ids.reshape(TOKENS, 1)
    out = pl.pallas_call(
        _ln_tc_kernel,
        grid=(ngrid,),
        in_specs=[
            pl.BlockSpec((TC_ROWS, H), lambda i: (i, 0)),
            pl.BlockSpec((TC_ROWS, 1), lambda i: (i, 0)),
            pl.BlockSpec((2, H), lambda i: (0, 0)),
            pl.BlockSpec((1, H), lambda i: (0, 0)),
            pl.BlockSpec((1, H), lambda i: (0, 0)),
        ],
        out_specs=pl.BlockSpec((TC_ROWS, H), lambda i: (i, 0)),
        out_shape=jax.ShapeDtypeStruct((TOKENS, H), jnp.float32),
    )(summed, tids, type_table, ln_gamma.reshape(1, H),
      ln_beta.reshape(1, H))
    return out.reshape(B, S, H)


def kernel(word_ids, pos_ids, type_ids, word_table, pos_table, type_table,
           ln_gamma, ln_beta):
    return _run(word_ids, pos_ids, type_ids, word_table, pos_table,
                type_table, ln_gamma, ln_beta)


# speedup vs baseline: 1.1114x; 1.0332x over previous
import functools

import jax
import jax.numpy as jnp
from jax import lax
from jax.experimental import pallas as pl
from jax.experimental.pallas import tpu as pltpu
from jax.experimental.pallas import tpu_sc as plsc

H = 128
L = 16
NC = 2
NS = 16
NW = NC * NS
B, S = 4, 4096
TOKENS = B * S
TOK_PER_W = TOKENS // NW
CHUNK = 128
NCHUNK = TOK_PER_W // CHUNK
EPS = 1e-6

TC_ROWS = 4096


def _gather_sum_body(wid_hbm, pid_hbm, wtab_hbm, ptab_hbm, out_hbm,
                     idxw_v, idxp_v, rows_w, rows_p, semw, semp):
    w = lax.axis_index("s") * NC + lax.axis_index("c")
    base_w = w * TOK_PER_W
    b_idx = w // (S // TOK_PER_W)
    col0 = (w % (S // TOK_PER_W)) * TOK_PER_W

    pltpu.sync_copy(wid_hbm.at[b_idx, pl.ds(col0, TOK_PER_W)], idxw_v)
    pltpu.sync_copy(pid_hbm.at[b_idx, pl.ds(col0, TOK_PER_W)], idxp_v)

    def start(c):
        b = c % 2
        cw = pltpu.async_copy(
            wtab_hbm.at[idxw_v.at[pl.ds(c * CHUNK, CHUNK)]], rows_w.at[b],
            semw)
        cp = pltpu.async_copy(
            ptab_hbm.at[idxp_v.at[pl.ds(c * CHUNK, CHUNK)]], rows_p.at[b],
            semp)
        return cw, cp

    pend = start(0)
    for c in range(NCHUNK):
        b = c % 2
        cw, cp = pend
        cw.wait()
        cp.wait()
        if c + 1 < NCHUNK:
            pend = start(c + 1)

        def body(t, carry):
            for j in range(H // L):
                sl = pl.ds(j * L, L)
                rows_w[b, t, sl] = rows_w[b, t, sl] + rows_p[b, t, sl]
            return carry

        lax.fori_loop(0, CHUNK, body, 0, unroll=4)
        pltpu.sync_copy(rows_w.at[b],
                        out_hbm.at[pl.ds(base_w + c * CHUNK, CHUNK)])


def _ln_tc_kernel(sum_ref, tid_ref, ttab_ref, gamma_ref, beta_ref, out_ref):
    x = sum_ref[...]
    tidf = tid_ref[...].astype(jnp.float32)
    t0 = ttab_ref[0:1, :]
    t1 = ttab_ref[1:2, :]
    x = x + t0 + tidf * (t1 - t0)
    mean = jnp.mean(x, axis=-1, keepdims=True)
    var = jnp.mean(jnp.square(x - mean), axis=-1, keepdims=True)
    normed = (x - mean) * lax.rsqrt(var + EPS)
    out_ref[0, ...] = normed * gamma_ref[0, :] + beta_ref[0, :]


@jax.jit
def _run(word_ids, pos_ids, type_ids, word_table, pos_table, type_table,
         ln_gamma, ln_beta):
    mesh = plsc.VectorSubcoreMesh(core_axis_name="c", subcore_axis_name="s")
    sc_k = pl.kernel(
        _gather_sum_body,
        mesh=mesh,
        compiler_params=pltpu.CompilerParams(needs_layout_passes=False),
        out_type=jax.ShapeDtypeStruct((TOKENS, H), jnp.float32),
        scratch_types=[
            pltpu.VMEM((TOK_PER_W,), jnp.int32),
            pltpu.VMEM((TOK_PER_W,), jnp.int32),
            pltpu.VMEM((2, CHUNK, H), jnp.float32),
            pltpu.VMEM((2, CHUNK, H), jnp.float32),
            pltpu.SemaphoreType.DMA,
            pltpu.SemaphoreType.DMA,
        ],
    )
    summed = sc_k(word_ids, pos_ids, word_table, pos_table)

    ngrid = TOKENS // TC_ROWS
    tids = type_ids.reshape(TOKENS, 1)
    out = pl.pallas_call(
        _ln_tc_kernel,
        grid=(ngrid,),
        in_specs=[
            pl.BlockSpec((TC_ROWS, H), lambda i: (i, 0)),
            pl.BlockSpec((TC_ROWS, 1), lambda i: (i, 0)),
            pl.BlockSpec((2, H), lambda i: (0, 0)),
            pl.BlockSpec((1, H), lambda i: (0, 0)),
            pl.BlockSpec((1, H), lambda i: (0, 0)),
        ],
        out_specs=pl.BlockSpec((1, TC_ROWS, H),
                               lambda i: (i // (S // TC_ROWS),
                                          i % (S // TC_ROWS), 0)),
        out_shape=jax.ShapeDtypeStruct((B, S, H), jnp.float32),
    )(summed, tids, type_table, ln_gamma.reshape(1, H),
      ln_beta.reshape(1, H))
    return out


def kernel(word_ids, pos_ids, type_ids, word_table, pos_table, type_table,
           ln_gamma, ln_beta):
    return _run(word_ids, pos_ids, type_ids, word_table, pos_table,
                type_table, ln_gamma, ln_beta)


# speedup vs baseline: 1.1362x; 1.0223x over previous
import functools

import jax
import jax.numpy as jnp
from jax import lax
from jax.experimental import pallas as pl
from jax.experimental.pallas import tpu as pltpu
from jax.experimental.pallas import tpu_sc as plsc

H = 128
L = 16
NC = 2
NS = 16
NW = NC * NS
B, S = 4, 4096
TOKENS = B * S
TOK_PER_W = TOKENS // NW
CHUNK = 128
NCHUNK = TOK_PER_W // CHUNK
EPS = 1e-6

TC_ROWS = 4096


NBUF = 3


def _gather_sum_body(wid_hbm, pid_hbm, wtab_hbm, ptab_hbm, out_hbm,
                     idxw_v, idxp_v, rows_w, rows_p, semw, semp, semo):
    w = lax.axis_index("s") * NC + lax.axis_index("c")
    base_w = w * TOK_PER_W
    b_idx = w // (S // TOK_PER_W)
    col0 = (w % (S // TOK_PER_W)) * TOK_PER_W

    pltpu.sync_copy(wid_hbm.at[b_idx, pl.ds(col0, TOK_PER_W)], idxw_v)
    pltpu.sync_copy(pid_hbm.at[b_idx, pl.ds(col0, TOK_PER_W)], idxp_v)

    def start(c):
        b = c % NBUF
        cw = pltpu.async_copy(
            wtab_hbm.at[idxw_v.at[pl.ds(c * CHUNK, CHUNK)]], rows_w.at[b],
            semw)
        cp = pltpu.async_copy(
            ptab_hbm.at[idxp_v.at[pl.ds(c * CHUNK, CHUNK)]], rows_p.at[b],
            semp)
        return cw, cp

    pend = [start(0), start(1)]
    wbs = []
    for c in range(NCHUNK):
        b = c % NBUF
        cw, cp = pend.pop(0)
        cw.wait()
        cp.wait()
        if c + 2 < NCHUNK:
            if wbs:
                wbs.pop(0).wait()
            pend.append(start(c + 2))

        def body(t, carry):
            for j in range(H // L):
                sl = pl.ds(j * L, L)
                rows_w[b, t, sl] = rows_w[b, t, sl] + rows_p[b, t, sl]
            return carry

        lax.fori_loop(0, CHUNK, body, 0, unroll=4)
        wbs.append(pltpu.async_copy(
            rows_w.at[b], out_hbm.at[pl.ds(base_w + c * CHUNK, CHUNK)],
            semo))
    for wb in wbs:
        wb.wait()


def _ln_tc_kernel(sum_ref, tid_ref, ttab_ref, gamma_ref, beta_ref, out_ref):
    x = sum_ref[...]
    tidf = tid_ref[...].astype(jnp.float32)
    t0 = ttab_ref[0:1, :]
    t1 = ttab_ref[1:2, :]
    x = x + t0 + tidf * (t1 - t0)
    mean = jnp.mean(x, axis=-1, keepdims=True)
    var = jnp.mean(jnp.square(x - mean), axis=-1, keepdims=True)
    normed = (x - mean) * lax.rsqrt(var + EPS)
    out_ref[0, ...] = normed * gamma_ref[0, :] + beta_ref[0, :]


@jax.jit
def _run(word_ids, pos_ids, type_ids, word_table, pos_table, type_table,
         ln_gamma, ln_beta):
    mesh = plsc.VectorSubcoreMesh(core_axis_name="c", subcore_axis_name="s")
    sc_k = pl.kernel(
        _gather_sum_body,
        mesh=mesh,
        compiler_params=pltpu.CompilerParams(needs_layout_passes=False),
        out_type=jax.ShapeDtypeStruct((TOKENS, H), jnp.float32),
        scratch_types=[
            pltpu.VMEM((TOK_PER_W,), jnp.int32),
            pltpu.VMEM((TOK_PER_W,), jnp.int32),
            pltpu.VMEM((NBUF, CHUNK, H), jnp.float32),
            pltpu.VMEM((NBUF, CHUNK, H), jnp.float32),
            pltpu.SemaphoreType.DMA,
            pltpu.SemaphoreType.DMA,
            pltpu.SemaphoreType.DMA,
        ],
    )
    summed = sc_k(word_ids, pos_ids, word_table, pos_table)

    ngrid = TOKENS // TC_ROWS
    tids = type_ids.reshape(TOKENS, 1)
    out = pl.pallas_call(
        _ln_tc_kernel,
        grid=(ngrid,),
        in_specs=[
            pl.BlockSpec((TC_ROWS, H), lambda i: (i, 0)),
            pl.BlockSpec((TC_ROWS, 1), lambda i: (i, 0)),
            pl.BlockSpec((2, H), lambda i: (0, 0)),
            pl.BlockSpec((1, H), lambda i: (0, 0)),
            pl.BlockSpec((1, H), lambda i: (0, 0)),
        ],
        out_specs=pl.BlockSpec((1, TC_ROWS, H),
                               lambda i: (i // (S // TC_ROWS),
                                          i % (S // TC_ROWS), 0)),
        out_shape=jax.ShapeDtypeStruct((B, S, H), jnp.float32),
    )(summed, tids, type_table, ln_gamma.reshape(1, H),
      ln_beta.reshape(1, H))
    return out


def kernel(word_ids, pos_ids, type_ids, word_table, pos_table, type_table,
           ln_gamma, ln_beta):
    return _run(word_ids, pos_ids, type_ids, word_table, pos_table,
                type_table, ln_gamma, ln_beta)
